# baseline (speedup 1.0000x reference)
import functools

import jax
import jax.numpy as jnp
from jax import lax
from jax.experimental import pallas as pl
from jax.experimental.pallas import tpu as pltpu
from jax.experimental.pallas import tpu_sc as plsc

P_NODES = 8192
A_NODES = 8192
NG = 262144
NCORR = 131072

NUM_SC = 2
NUM_SUBCORES = 16
NW = NUM_SC * NUM_SUBCORES
LANES = 16

MAP_SIZE = P_NODES * A_NODES
MEMSET_CHUNK = 1 << 20
MAP_PAD = MAP_SIZE + MEMSET_CHUNK
TRASH = MAP_SIZE

G_PER_W = NG // NW
G_ROWS = G_PER_W // 128
Q_PER_W = NCORR // NW
Q_ROWS = Q_PER_W // 128

ACCEPTANCE_OVERLAP = 0.1
ACCEPTANCE_RADIUS = 0.1
RRE_THRESHOLD = 15.0
RTE_THRESHOLD = 0.3

_SC_MESH = plsc.VectorSubcoreMesh(
    core_axis_name="c", subcore_axis_name="s",
    num_cores=NUM_SC, num_subcores=NUM_SUBCORES)


def _memset_body(o_ref):
    o_ref[...] = jnp.zeros_like(o_ref)


_zero_map = pl.pallas_call(
    _memset_body,
    grid=(MAP_PAD // MEMSET_CHUNK,),
    out_specs=pl.BlockSpec((MEMSET_CHUNK,), lambda i: (i,)),
    out_shape=jax.ShapeDtypeStruct((MAP_PAD,), jnp.float32),
)


@functools.partial(
    pl.kernel,
    out_type=jax.ShapeDtypeStruct((NW * LANES,), jnp.float32),
    mesh=_SC_MESH,
    scratch_types=[
        pltpu.VMEM((G_PER_W,), jnp.int32),
        pltpu.VMEM((G_PER_W,), jnp.int32),
        pltpu.VMEM((G_PER_W,), jnp.float32),
        pltpu.VMEM((G_ROWS, 128), jnp.int32),
        pltpu.VMEM((G_ROWS, 128), jnp.float32),
        pltpu.SemaphoreType.DMA,
    ],
    compiler_params=pltpu.CompilerParams(has_side_effects=True),
)
def _sc_scatter(gt_pos_hbm, gt_anc_hbm, ov_hbm, map_hbm, tok_hbm,
                posv, ancv, ovv, keys2d, ones2d, sem):
    wid = lax.axis_index("s") * NUM_SC + lax.axis_index("c")
    base = wid * G_PER_W
    pltpu.sync_copy(gt_pos_hbm.at[pl.ds(base, G_PER_W)], posv)
    pltpu.sync_copy(gt_anc_hbm.at[pl.ds(base, G_PER_W)], ancv)
    pltpu.sync_copy(ov_hbm.at[pl.ds(base, G_PER_W)], ovv)


    lane_iota = lax.iota(jnp.int32, LANES)

    def key_row(j, carry):
        for c in range(8):
            off = j * 128 + c * LANES
            p = posv[pl.ds(off, LANES)]
            a = ancv[pl.ds(off, LANES)]
            o = ovv[pl.ds(off, LANES)]
            trash = (TRASH + base + off) + lane_iota
            key = jnp.where(o > ACCEPTANCE_OVERLAP, p * A_NODES + a, trash)
            keys2d[j, pl.ds(c * LANES, LANES)] = key
            ones2d[j, pl.ds(c * LANES, LANES)] = jnp.full((LANES,), 1.0,
                                                          jnp.float32)
        return carry

    lax.fori_loop(0, G_ROWS, key_row, 0)

    def issue(j, carry):
        pltpu.async_copy(ones2d.at[j], map_hbm.at[keys2d.at[j]], sem)
        return carry

    lax.fori_loop(0, G_ROWS, issue, 0)

    def drain(j, carry):
        pltpu.make_async_copy(ones2d.at[0], map_hbm.at[keys2d.at[0]], sem).wait()
        return carry

    lax.fori_loop(0, G_ROWS, drain, 0)

    pltpu.sync_copy(ones2d.at[0].at[pl.ds(0, LANES)],
                    tok_hbm.at[pl.ds(wid * LANES, LANES)])


@functools.partial(
    pl.kernel,
    out_type=jax.ShapeDtypeStruct((NW * LANES,), jnp.float32),
    mesh=_SC_MESH,
    scratch_types=[
        pltpu.VMEM((Q_PER_W,), jnp.int32),
        pltpu.VMEM((Q_PER_W,), jnp.int32),
        pltpu.VMEM((Q_PER_W,), jnp.int32),
        pltpu.VMEM((Q_PER_W,), jnp.float32),
        pltpu.VMEM((LANES,), jnp.float32),
        pltpu.SemaphoreType.DMA,
    ],
)
def _sc_gather(map_hbm, qpos_hbm, qanc_hbm, tok_hbm, out_hbm,
               pidx, aidx, qkeys, vals, accv, sem):
    wid = lax.axis_index("s") * NUM_SC + lax.axis_index("c")
    base = wid * Q_PER_W
    pltpu.sync_copy(qpos_hbm.at[pl.ds(base, Q_PER_W)], pidx)
    pltpu.sync_copy(qanc_hbm.at[pl.ds(base, Q_PER_W)], aidx)

    def key_step(i, carry):
        off = i * LANES
        q = pidx[pl.ds(off, LANES)] * A_NODES + aidx[pl.ds(off, LANES)]
        qkeys[pl.ds(off, LANES)] = q
        return carry

    lax.fori_loop(0, Q_PER_W // LANES, key_step, 0)

    def issue(j, carry):
        pltpu.async_copy(map_hbm.at[qkeys.at[pl.ds(j * 128, 128)]],
                         vals.at[pl.ds(j * 128, 128)], sem)
        return carry

    lax.fori_loop(0, Q_ROWS, issue, 0)

    def drain(j, carry):
        pltpu.make_async_copy(map_hbm.at[qkeys.at[pl.ds(0, 128)]],
                              vals.at[pl.ds(0, 128)], sem).wait()
        return carry

    lax.fori_loop(0, Q_ROWS, drain, 0)

    def acc_step(i, acc):
        return acc + vals[pl.ds(i * LANES, LANES)]

    acc = lax.fori_loop(0, Q_PER_W // LANES, acc_step,
                        jnp.zeros((LANES,), jnp.float32))
    accv[...] = acc
    pltpu.sync_copy(accv, out_hbm.at[pl.ds(wid * LANES, LANES)])


_FINE_CHUNK = 16384
_FINE_STEPS = NCORR // _FINE_CHUNK


def _finalize_body(part_ref, pos_ref, anc_ref, t_ref, e_ref,
                   c_ref, f_ref, rre_ref, rte_ref, rec_ref):
    i = pl.program_id(0)

    tb = t_ref[...]
    rb = tb.astype(jnp.bfloat16).astype(jnp.float32)

    def b16(v):
        return v.astype(jnp.bfloat16).astype(jnp.float32)

    ax = b16(anc_ref[0, :])
    ay = b16(anc_ref[1, :])
    az = b16(anc_ref[2, :])
    wx = rb[0, 0] * ax + rb[0, 1] * ay + rb[0, 2] * az + tb[0, 3]
    wy = rb[1, 0] * ax + rb[1, 1] * ay + rb[1, 2] * az + tb[1, 3]
    wz = rb[2, 0] * ax + rb[2, 1] * ay + rb[2, 2] * az + tb[2, 3]
    dx = pos_ref[0, :] - wx
    dy = pos_ref[1, :] - wy
    dz = pos_ref[2, :] - wz
    dist = jnp.sqrt(dx * dx + dy * dy + dz * dz)
    cnt = jnp.sum((dist < ACCEPTANCE_RADIUS).astype(jnp.float32))

    @pl.when(i == 0)
    def _():
        f_ref[...] = jnp.zeros((1, 1), jnp.float32)
        c_ref[...] = jnp.full((1, 1), jnp.sum(part_ref[...]) * (1.0 / NCORR))
        eb = e_ref[...]
        tr = jnp.sum(tb[:3, :3] * eb[:3, :3])
        x = jnp.clip(0.5 * (tr - 1.0), -1.0, 1.0)
        rre = jnp.arctan2(jnp.sqrt(jnp.maximum((1.0 + x) * (1.0 - x), 0.0)),
                          x) * (180.0 / jnp.pi)
        dt = tb[:3, 3] - eb[:3, 3]
        rte = jnp.sqrt(jnp.sum(dt * dt))
        rre_ref[...] = jnp.full((1, 1), rre)
        rte_ref[...] = jnp.full((1, 1), rte)
        rec_ref[...] = jnp.where(
            jnp.logical_and(rre < RRE_THRESHOLD, rte < RTE_THRESHOLD),
            jnp.ones((1, 1), jnp.float32), jnp.zeros((1, 1), jnp.float32))

    f_ref[...] = f_ref[...] + cnt

    @pl.when(i == pl.num_programs(0) - 1)
    def _():
        f_ref[...] = f_ref[...] * (1.0 / NCORR)


_scalar_spec = pl.BlockSpec((1, 1), lambda i: (0, 0))
_finalize = pl.pallas_call(
    _finalize_body,
    grid=(_FINE_STEPS,),
    in_specs=[
        pl.BlockSpec((NW * LANES,), lambda i: (0,)),
        pl.BlockSpec((3, _FINE_CHUNK), lambda i: (0, i)),
        pl.BlockSpec((3, _FINE_CHUNK), lambda i: (0, i)),
        pl.BlockSpec((4, 4), lambda i: (0, 0)),
        pl.BlockSpec((4, 4), lambda i: (0, 0)),
    ],
    out_specs=[_scalar_spec] * 5,
    out_shape=[jax.ShapeDtypeStruct((1, 1), jnp.float32)] * 5,
)


def kernel(pos_points_c, anc_points_c, gt_node_corr_overlaps,
           gt_node_corr_indices, pos_node_corr_indices,
           anc_node_corr_indices, pos_corr_points, anc_corr_points,
           transform, est_transform):
    del pos_points_c, anc_points_c
    gt_pos = gt_node_corr_indices[:, 0]
    gt_anc = gt_node_corr_indices[:, 1]

    corr_map = _zero_map()
    tok = _sc_scatter(gt_pos, gt_anc, gt_node_corr_overlaps, corr_map)
    partials = _sc_gather(corr_map, pos_node_corr_indices,
                          anc_node_corr_indices, tok)

    pos_t = pos_corr_points.T
    anc_t = anc_corr_points.T
    c_p, f_p, rre, rte, rec = _finalize(partials, pos_t, anc_t,
                                        transform, est_transform)
    return (c_p[0, 0], f_p[0, 0], rre[0, 0], rte[0, 0], rec[0, 0])


# speedup vs baseline: 1.0132x; 1.0132x over previous
import functools

import jax
import jax.numpy as jnp
from jax import lax
from jax.experimental import pallas as pl
from jax.experimental.pallas import tpu as pltpu
from jax.experimental.pallas import tpu_sc as plsc

P_NODES = 8192
A_NODES = 8192
NG = 262144
NCORR = 131072

NUM_SC = 2
NUM_SUBCORES = 16
NW = NUM_SC * NUM_SUBCORES
LANES = 16

MAP_SIZE = P_NODES * A_NODES
MEMSET_CHUNK = 1 << 22
MAP_PAD = MAP_SIZE + MEMSET_CHUNK
TRASH = MAP_SIZE

G_PER_W = NG // NW
G_ROWS = G_PER_W // 128
Q_PER_W = NCORR // NW
Q_ROWS = Q_PER_W // 128

ACCEPTANCE_OVERLAP = 0.1
ACCEPTANCE_RADIUS = 0.1
RRE_THRESHOLD = 15.0
RTE_THRESHOLD = 0.3

_SC_MESH = plsc.VectorSubcoreMesh(
    core_axis_name="c", subcore_axis_name="s",
    num_cores=NUM_SC, num_subcores=NUM_SUBCORES)


def _memset_body(o_ref):
    o_ref[...] = jnp.zeros_like(o_ref)


_zero_map = pl.pallas_call(
    _memset_body,
    grid=(MAP_PAD // MEMSET_CHUNK,),
    out_specs=pl.BlockSpec((MEMSET_CHUNK,), lambda i: (i,)),
    out_shape=jax.ShapeDtypeStruct((MAP_PAD,), jnp.float32),
)


@functools.partial(
    pl.kernel,
    out_type=jax.ShapeDtypeStruct((NW * LANES,), jnp.float32),
    mesh=_SC_MESH,
    scratch_types=[
        pltpu.VMEM((G_PER_W,), jnp.int32),
        pltpu.VMEM((G_PER_W,), jnp.int32),
        pltpu.VMEM((G_PER_W,), jnp.float32),
        pltpu.VMEM((G_ROWS, 128), jnp.int32),
        pltpu.VMEM((128,), jnp.float32),
        pltpu.VMEM((LANES,), jnp.float32),
        pltpu.SemaphoreType.DMA,
    ],
    compiler_params=pltpu.CompilerParams(has_side_effects=True),
)
def _sc_scatter(gt_pos_hbm, gt_anc_hbm, ov_hbm, map_hbm, tok_hbm,
                posv, ancv, ovv, keys2d, ones_v, tokv, sem):
    wid = lax.axis_index("s") * NUM_SC + lax.axis_index("c")
    base = wid * G_PER_W
    pltpu.sync_copy(gt_pos_hbm.at[pl.ds(base, G_PER_W)], posv)
    pltpu.sync_copy(gt_anc_hbm.at[pl.ds(base, G_PER_W)], ancv)
    pltpu.sync_copy(ov_hbm.at[pl.ds(base, G_PER_W)], ovv)


    lane_iota = lax.iota(jnp.int32, LANES)
    for c in range(8):
        ones_v[pl.ds(c * LANES, LANES)] = jnp.full((LANES,), 1.0, jnp.float32)

    def key_row(j, carry):
        for c in range(8):
            off = j * 128 + c * LANES
            p = posv[pl.ds(off, LANES)]
            a = ancv[pl.ds(off, LANES)]
            o = ovv[pl.ds(off, LANES)]
            trash = (TRASH + base + off) + lane_iota
            key = jnp.where(o > ACCEPTANCE_OVERLAP, p * A_NODES + a, trash)
            keys2d[j, pl.ds(c * LANES, LANES)] = key
        return carry

    lax.fori_loop(0, G_ROWS, key_row, 0)

    def issue(j, carry):
        pltpu.async_copy(ones_v, map_hbm.at[keys2d.at[j]], sem)
        return carry

    lax.fori_loop(0, G_ROWS, issue, 0)

    def drain(j, carry):
        pltpu.make_async_copy(ones_v, map_hbm.at[keys2d.at[0]], sem).wait()
        return carry

    lax.fori_loop(0, G_ROWS, drain, 0)

    tokv[...] = jnp.full((LANES,), 1.0, jnp.float32)
    pltpu.sync_copy(tokv, tok_hbm.at[pl.ds(wid * LANES, LANES)])


@functools.partial(
    pl.kernel,
    out_type=jax.ShapeDtypeStruct((NW * LANES,), jnp.float32),
    mesh=_SC_MESH,
    scratch_types=[
        pltpu.VMEM((Q_PER_W,), jnp.int32),
        pltpu.VMEM((Q_PER_W,), jnp.int32),
        pltpu.VMEM((Q_PER_W,), jnp.int32),
        pltpu.VMEM((Q_PER_W,), jnp.float32),
        pltpu.VMEM((LANES,), jnp.float32),
        pltpu.SemaphoreType.DMA,
    ],
)
def _sc_gather(map_hbm, qpos_hbm, qanc_hbm, tok_hbm, out_hbm,
               pidx, aidx, qkeys, vals, accv, sem):
    wid = lax.axis_index("s") * NUM_SC + lax.axis_index("c")
    base = wid * Q_PER_W
    pltpu.sync_copy(qpos_hbm.at[pl.ds(base, Q_PER_W)], pidx)
    pltpu.sync_copy(qanc_hbm.at[pl.ds(base, Q_PER_W)], aidx)

    def key_step(i, carry):
        off = i * LANES
        q = pidx[pl.ds(off, LANES)] * A_NODES + aidx[pl.ds(off, LANES)]
        qkeys[pl.ds(off, LANES)] = q
        return carry

    lax.fori_loop(0, Q_PER_W // LANES, key_step, 0)

    def issue(j, carry):
        pltpu.async_copy(map_hbm.at[qkeys.at[pl.ds(j * 128, 128)]],
                         vals.at[pl.ds(j * 128, 128)], sem)
        return carry

    lax.fori_loop(0, Q_ROWS, issue, 0)

    def drain(j, carry):
        pltpu.make_async_copy(map_hbm.at[qkeys.at[pl.ds(0, 128)]],
                              vals.at[pl.ds(0, 128)], sem).wait()
        return carry

    lax.fori_loop(0, Q_ROWS, drain, 0)

    def acc_step(i, acc):
        return acc + vals[pl.ds(i * LANES, LANES)]

    acc = lax.fori_loop(0, Q_PER_W // LANES, acc_step,
                        jnp.zeros((LANES,), jnp.float32))
    accv[...] = acc
    pltpu.sync_copy(accv, out_hbm.at[pl.ds(wid * LANES, LANES)])


_FINE_CHUNK = 16384
_FINE_STEPS = NCORR // _FINE_CHUNK


def _finalize_body(pos_ref, anc_ref, t_ref, e_ref,
                   f_ref, rre_ref, rte_ref, rec_ref):
    i = pl.program_id(0)

    tb = t_ref[...]
    rb = tb.astype(jnp.bfloat16).astype(jnp.float32)

    def b16(v):
        return v.astype(jnp.bfloat16).astype(jnp.float32)

    ax = b16(anc_ref[0, :])
    ay = b16(anc_ref[1, :])
    az = b16(anc_ref[2, :])
    wx = rb[0, 0] * ax + rb[0, 1] * ay + rb[0, 2] * az + tb[0, 3]
    wy = rb[1, 0] * ax + rb[1, 1] * ay + rb[1, 2] * az + tb[1, 3]
    wz = rb[2, 0] * ax + rb[2, 1] * ay + rb[2, 2] * az + tb[2, 3]
    dx = pos_ref[0, :] - wx
    dy = pos_ref[1, :] - wy
    dz = pos_ref[2, :] - wz
    dist = jnp.sqrt(dx * dx + dy * dy + dz * dz)
    cnt = jnp.sum((dist < ACCEPTANCE_RADIUS).astype(jnp.float32))

    @pl.when(i == 0)
    def _():
        f_ref[...] = jnp.zeros((1, 1), jnp.float32)
        eb = e_ref[...]
        tr = jnp.sum(tb[:3, :3] * eb[:3, :3])
        x = jnp.clip(0.5 * (tr - 1.0), -1.0, 1.0)
        rre = jnp.arctan2(jnp.sqrt(jnp.maximum((1.0 + x) * (1.0 - x), 0.0)),
                          x) * (180.0 / jnp.pi)
        dt = tb[:3, 3] - eb[:3, 3]
        rte = jnp.sqrt(jnp.sum(dt * dt))
        rre_ref[...] = jnp.full((1, 1), rre)
        rte_ref[...] = jnp.full((1, 1), rte)
        rec_ref[...] = jnp.where(
            jnp.logical_and(rre < RRE_THRESHOLD, rte < RTE_THRESHOLD),
            jnp.ones((1, 1), jnp.float32), jnp.zeros((1, 1), jnp.float32))

    f_ref[...] = f_ref[...] + cnt

    @pl.when(i == pl.num_programs(0) - 1)
    def _():
        f_ref[...] = f_ref[...] * (1.0 / NCORR)


_scalar_spec = pl.BlockSpec((1, 1), lambda i: (0, 0))
_finalize = pl.pallas_call(
    _finalize_body,
    grid=(_FINE_STEPS,),
    in_specs=[
        pl.BlockSpec((3, _FINE_CHUNK), lambda i: (0, i)),
        pl.BlockSpec((3, _FINE_CHUNK), lambda i: (0, i)),
        pl.BlockSpec((4, 4), lambda i: (0, 0)),
        pl.BlockSpec((4, 4), lambda i: (0, 0)),
    ],
    out_specs=[_scalar_spec] * 4,
    out_shape=[jax.ShapeDtypeStruct((1, 1), jnp.float32)] * 4,
)


def _combine_body(part_ref, c_ref):
    c_ref[...] = jnp.full((1, 1), jnp.sum(part_ref[...]) * (1.0 / NCORR))


_combine = pl.pallas_call(
    _combine_body,
    in_specs=[pl.BlockSpec((NW * LANES,), lambda: (0,))],
    out_specs=pl.BlockSpec((1, 1), lambda: (0, 0)),
    out_shape=jax.ShapeDtypeStruct((1, 1), jnp.float32),
)


def kernel(pos_points_c, anc_points_c, gt_node_corr_overlaps,
           gt_node_corr_indices, pos_node_corr_indices,
           anc_node_corr_indices, pos_corr_points, anc_corr_points,
           transform, est_transform):
    del pos_points_c, anc_points_c
    gt_pos = gt_node_corr_indices[:, 0]
    gt_anc = gt_node_corr_indices[:, 1]

    corr_map = _zero_map()
    tok = _sc_scatter(gt_pos, gt_anc, gt_node_corr_overlaps, corr_map)

    pos_t = pos_corr_points.T
    anc_t = anc_corr_points.T
    f_p, rre, rte, rec = _finalize(pos_t, anc_t, transform, est_transform)

    partials = _sc_gather(corr_map, pos_node_corr_indices,
                          anc_node_corr_indices, tok)
    c_p = _combine(partials)
    return (c_p[0, 0], f_p[0, 0], rre[0, 0], rte[0, 0], rec[0, 0])


# speedup vs baseline: 1.0164x; 1.0032x over previous
import functools

import jax
import jax.numpy as jnp
from jax import lax
from jax.experimental import pallas as pl
from jax.experimental.pallas import tpu as pltpu
from jax.experimental.pallas import tpu_sc as plsc

P_NODES = 8192
A_NODES = 8192
NG = 262144
NCORR = 131072

NUM_SC = 2
NUM_SUBCORES = 16
NW = NUM_SC * NUM_SUBCORES
LANES = 16

MAP_SIZE = P_NODES * A_NODES
MEMSET_CHUNK = 1 << 22
MAP_PAD = MAP_SIZE + MEMSET_CHUNK
TRASH = MAP_SIZE

G_PER_W = NG // NW
G_ROWS = G_PER_W // 128
Q_PER_W = NCORR // NW
Q_ROWS = Q_PER_W // 128

ACCEPTANCE_OVERLAP = 0.1
ACCEPTANCE_RADIUS = 0.1
RRE_THRESHOLD = 15.0
RTE_THRESHOLD = 0.3

_SC_MESH = plsc.VectorSubcoreMesh(
    core_axis_name="c", subcore_axis_name="s",
    num_cores=NUM_SC, num_subcores=NUM_SUBCORES)


def _memset_body(o_ref):
    o_ref[...] = jnp.zeros_like(o_ref)


_zero_map = pl.pallas_call(
    _memset_body,
    grid=(MAP_PAD // MEMSET_CHUNK,),
    out_specs=pl.BlockSpec((MEMSET_CHUNK,), lambda i: (i,)),
    out_shape=jax.ShapeDtypeStruct((MAP_PAD,), jnp.float32),
)


@functools.partial(
    pl.kernel,
    out_type=jax.ShapeDtypeStruct((NW * LANES,), jnp.float32),
    mesh=_SC_MESH,
    scratch_types=[
        pltpu.VMEM((G_PER_W,), jnp.int32),
        pltpu.VMEM((G_PER_W,), jnp.int32),
        pltpu.VMEM((G_PER_W,), jnp.float32),
        pltpu.VMEM((G_ROWS, 128), jnp.int32),
        pltpu.VMEM((128,), jnp.float32),
        pltpu.VMEM((LANES,), jnp.float32),
        pltpu.SemaphoreType.DMA,
    ],
    compiler_params=pltpu.CompilerParams(has_side_effects=True),
)
def _sc_scatter(gt_pos_hbm, gt_anc_hbm, ov_hbm, map_hbm, tok_hbm,
                posv, ancv, ovv, keys2d, ones_v, tokv, sem):
    wid = lax.axis_index("s") * NUM_SC + lax.axis_index("c")
    base = wid * G_PER_W
    pltpu.sync_copy(gt_pos_hbm.at[pl.ds(base, G_PER_W)], posv)
    pltpu.sync_copy(gt_anc_hbm.at[pl.ds(base, G_PER_W)], ancv)
    pltpu.sync_copy(ov_hbm.at[pl.ds(base, G_PER_W)], ovv)


    lane_iota = lax.iota(jnp.int32, LANES)
    for c in range(8):
        ones_v[pl.ds(c * LANES, LANES)] = jnp.full((LANES,), 1.0, jnp.float32)

    def key_row(j, carry):
        for c in range(8):
            off = j * 128 + c * LANES
            p = posv[pl.ds(off, LANES)]
            a = ancv[pl.ds(off, LANES)]
            o = ovv[pl.ds(off, LANES)]
            trash = (TRASH + base + off) + lane_iota
            key = jnp.where(o > ACCEPTANCE_OVERLAP, p * A_NODES + a, trash)
            keys2d[j, pl.ds(c * LANES, LANES)] = key
        return carry

    lax.fori_loop(0, G_ROWS, key_row, 0)

    def issue(j, carry):
        pltpu.async_copy(ones_v, map_hbm.at[keys2d.at[j]], sem)
        return carry

    lax.fori_loop(0, G_ROWS, issue, 0)

    def drain(j, carry):
        pltpu.make_async_copy(ones_v, map_hbm.at[keys2d.at[0]], sem).wait()
        return carry

    lax.fori_loop(0, G_ROWS, drain, 0)

    tokv[...] = jnp.full((LANES,), 1.0, jnp.float32)
    pltpu.sync_copy(tokv, tok_hbm.at[pl.ds(wid * LANES, LANES)])


@functools.partial(
    pl.kernel,
    out_type=jax.ShapeDtypeStruct((NW * LANES,), jnp.float32),
    mesh=_SC_MESH,
    scratch_types=[
        pltpu.VMEM((Q_PER_W,), jnp.int32),
        pltpu.VMEM((Q_PER_W,), jnp.int32),
        pltpu.VMEM((Q_PER_W,), jnp.int32),
        pltpu.VMEM((Q_PER_W,), jnp.float32),
        pltpu.VMEM((LANES,), jnp.float32),
        pltpu.SemaphoreType.DMA,
    ],
)
def _sc_gather(map_hbm, qpos_hbm, qanc_hbm, tok_hbm, out_hbm,
               pidx, aidx, qkeys, vals, accv, sem):
    wid = lax.axis_index("s") * NUM_SC + lax.axis_index("c")
    base = wid * Q_PER_W
    pltpu.sync_copy(qpos_hbm.at[pl.ds(base, Q_PER_W)], pidx)
    pltpu.sync_copy(qanc_hbm.at[pl.ds(base, Q_PER_W)], aidx)

    def key_step(i, carry):
        off = i * LANES
        q = pidx[pl.ds(off, LANES)] * A_NODES + aidx[pl.ds(off, LANES)]
        qkeys[pl.ds(off, LANES)] = q
        return carry

    lax.fori_loop(0, Q_PER_W // LANES, key_step, 0)

    def issue(j, carry):
        pltpu.async_copy(map_hbm.at[qkeys.at[pl.ds(j * 128, 128)]],
                         vals.at[pl.ds(j * 128, 128)], sem)
        return carry

    lax.fori_loop(0, Q_ROWS, issue, 0)

    def drain(j, carry):
        pltpu.make_async_copy(map_hbm.at[qkeys.at[pl.ds(0, 128)]],
                              vals.at[pl.ds(0, 128)], sem).wait()
        return carry

    lax.fori_loop(0, Q_ROWS, drain, 0)

    def acc_step(i, acc):
        return acc + vals[pl.ds(i * LANES, LANES)]

    acc = lax.fori_loop(0, Q_PER_W // LANES, acc_step,
                        jnp.zeros((LANES,), jnp.float32))
    accv[...] = acc
    pltpu.sync_copy(accv, out_hbm.at[pl.ds(wid * LANES, LANES)])


_FINE_CHUNK = 16384
_FINE_STEPS = NCORR // _FINE_CHUNK


def _finalize_body(pos_ref, anc_ref, t_ref, e_ref,
                   f_ref, rre_ref, rte_ref, rec_ref):
    i = pl.program_id(0)

    tb = t_ref[...]
    rb = tb.astype(jnp.bfloat16).astype(jnp.float32)

    def b16(v):
        return v.astype(jnp.bfloat16).astype(jnp.float32)

    ax = b16(anc_ref[0, :])
    ay = b16(anc_ref[1, :])
    az = b16(anc_ref[2, :])
    wx = rb[0, 0] * ax + rb[0, 1] * ay + rb[0, 2] * az + tb[0, 3]
    wy = rb[1, 0] * ax + rb[1, 1] * ay + rb[1, 2] * az + tb[1, 3]
    wz = rb[2, 0] * ax + rb[2, 1] * ay + rb[2, 2] * az + tb[2, 3]
    dx = pos_ref[0, :] - wx
    dy = pos_ref[1, :] - wy
    dz = pos_ref[2, :] - wz
    dist = jnp.sqrt(dx * dx + dy * dy + dz * dz)
    cnt = jnp.sum((dist < ACCEPTANCE_RADIUS).astype(jnp.float32))

    @pl.when(i == 0)
    def _():
        f_ref[...] = jnp.zeros((1, 1), jnp.float32)
        eb = e_ref[...]
        ebb = eb.astype(jnp.bfloat16).astype(jnp.float32)
        pr = rb * ebb
        d0 = (pr[0, 0] + pr[1, 0]) + pr[2, 0]
        d1 = (pr[0, 1] + pr[1, 1]) + pr[2, 1]
        d2 = (pr[0, 2] + pr[1, 2]) + pr[2, 2]
        tr = (d0 + d1) + d2
        x = jnp.clip(0.5 * (tr - 1.0), -1.0, 1.0)
        rre = jnp.where(
            x == -1.0,
            180.0,
            2.0 * jnp.arctan2(jnp.sqrt(jnp.maximum(1.0 - x * x, 0.0)),
                              1.0 + x) * (180.0 / jnp.pi))
        dt = tb[:3, 3] - eb[:3, 3]
        rte = jnp.sqrt(jnp.sum(dt * dt))
        rre_ref[...] = jnp.full((1, 1), rre)
        rte_ref[...] = jnp.full((1, 1), rte)
        rec_ref[...] = jnp.where(
            jnp.logical_and(rre < RRE_THRESHOLD, rte < RTE_THRESHOLD),
            jnp.ones((1, 1), jnp.float32), jnp.zeros((1, 1), jnp.float32))

    f_ref[...] = f_ref[...] + cnt

    @pl.when(i == pl.num_programs(0) - 1)
    def _():
        f_ref[...] = f_ref[...] * (1.0 / NCORR)


_scalar_spec = pl.BlockSpec((1, 1), lambda i: (0, 0))
_finalize = pl.pallas_call(
    _finalize_body,
    grid=(_FINE_STEPS,),
    in_specs=[
        pl.BlockSpec((3, _FINE_CHUNK), lambda i: (0, i)),
        pl.BlockSpec((3, _FINE_CHUNK), lambda i: (0, i)),
        pl.BlockSpec((4, 4), lambda i: (0, 0)),
        pl.BlockSpec((4, 4), lambda i: (0, 0)),
    ],
    out_specs=[_scalar_spec] * 4,
    out_shape=[jax.ShapeDtypeStruct((1, 1), jnp.float32)] * 4,
)


def _combine_body(part_ref, c_ref):
    c_ref[...] = jnp.full((1, 1), jnp.sum(part_ref[...]) * (1.0 / NCORR))


_combine = pl.pallas_call(
    _combine_body,
    in_specs=[pl.BlockSpec((NW * LANES,), lambda: (0,))],
    out_specs=pl.BlockSpec((1, 1), lambda: (0, 0)),
    out_shape=jax.ShapeDtypeStruct((1, 1), jnp.float32),
)


def kernel(pos_points_c, anc_points_c, gt_node_corr_overlaps,
           gt_node_corr_indices, pos_node_corr_indices,
           anc_node_corr_indices, pos_corr_points, anc_corr_points,
           transform, est_transform):
    del pos_points_c, anc_points_c
    gt_pos = gt_node_corr_indices[:, 0]
    gt_anc = gt_node_corr_indices[:, 1]

    corr_map = _zero_map()
    tok = _sc_scatter(gt_pos, gt_anc, gt_node_corr_overlaps, corr_map)

    pos_t = pos_corr_points.T
    anc_t = anc_corr_points.T
    f_p, rre, rte, rec = _finalize(pos_t, anc_t, transform, est_transform)

    partials = _sc_gather(corr_map, pos_node_corr_indices,
                          anc_node_corr_indices, tok)
    c_p = _combine(partials)
    return (c_p[0, 0], f_p[0, 0], rre[0, 0], rte[0, 0], rec[0, 0])
